# baseline (speedup 1.0000x reference)
import jax
import jax.numpy as jnp
from jax.experimental import pallas as pl
from jax.experimental.pallas import tpu as pltpu

_TB = 512


def _moe_body(inp_ref, rw_ref, rb_ref, w1_ref, b1_ref, w2_ref, b2_ref,
              out_ref, probs_ref):
    e = pl.program_id(1)
    t = inp_ref[0]

    @pl.when(e == 0)
    def _():
        logits = jnp.dot(t, rw_ref[...], preferred_element_type=jnp.float32)
        probs_ref[...] = jax.nn.softmax(logits + rb_ref[...], axis=-1)

    h = jnp.dot(t, w1_ref[0], preferred_element_type=jnp.float32)
    h = jax.nn.gelu(h + b1_ref[0])

    sel = jax.lax.broadcasted_iota(jnp.int32, (1, 8), 1) == e - 1
    gate = jnp.sum(probs_ref[...] * sel.astype(jnp.float32), axis=-1,
                   keepdims=True)
    gate = jnp.where(e == 0, 1.0, gate)

    hg = (h * gate).astype(jnp.bfloat16)
    contrib = jnp.dot(hg, w2_ref[0], preferred_element_type=jnp.float32)
    contrib = contrib + gate * b2_ref[0]

    @pl.when(e == 0)
    def _():
        out_ref[...] = contrib

    @pl.when(e > 0)
    def _():
        out_ref[...] += contrib


def kernel(x, embedding_tokens, router_W, router_b, expert_W1, expert_b1,
           expert_W2, expert_b2, shared_W1, shared_b1, shared_W2, shared_b2):
    B, S, D = x.shape
    F = shared_W1.shape[1]
    E = router_W.shape[1]
    n = B * S

    inp = jnp.stack([x.reshape(n, D), embedding_tokens.reshape(n, D)])
    inp = inp.astype(jnp.bfloat16)
    w1 = jnp.concatenate([shared_W1[None], expert_W1], 0).astype(jnp.bfloat16)
    w2 = jnp.concatenate([shared_W2[None], expert_W2], 0).astype(jnp.bfloat16)
    b1 = jnp.concatenate([shared_b1[None], expert_b1], 0)[:, None, :]
    b2 = jnp.concatenate([shared_b2[None], expert_b2], 0)[:, None, :]
    rw = router_W.astype(jnp.bfloat16)
    rb = router_b[None, :]

    out = pl.pallas_call(
        _moe_body,
        grid=(n // _TB, E + 1),
        in_specs=[
            pl.BlockSpec((1, _TB, D), lambda i, e: (jnp.minimum(e, 1), i, 0)),
            pl.BlockSpec((D, E), lambda i, e: (0, 0)),
            pl.BlockSpec((1, E), lambda i, e: (0, 0)),
            pl.BlockSpec((1, D, F), lambda i, e: (e, 0, 0)),
            pl.BlockSpec((1, 1, F), lambda i, e: (e, 0, 0)),
            pl.BlockSpec((1, F, D), lambda i, e: (e, 0, 0)),
            pl.BlockSpec((1, 1, D), lambda i, e: (e, 0, 0)),
        ],
        out_specs=pl.BlockSpec((_TB, D), lambda i, e: (i, 0)),
        out_shape=jax.ShapeDtypeStruct((n, D), jnp.float32),
        scratch_shapes=[pltpu.VMEM((_TB, E), jnp.float32)],
        compiler_params=pltpu.CompilerParams(
            dimension_semantics=("parallel", "arbitrary")),
    )(inp, rw, rb, w1, b1, w2, b2)

    return out.reshape(B, S, D)


# speedup vs baseline: 1.1523x; 1.1523x over previous
import jax
import jax.numpy as jnp
from jax.experimental import pallas as pl
from jax.experimental.pallas import tpu as pltpu

_TB = 512


def _moe_body(x_ref, emb_ref, rw_ref, rb_ref, w1_ref, b1_ref, w2_ref, b2_ref,
              sw1_ref, sb1_ref, sw2_ref, sb2_ref, out_ref, probs_ref):
    e = pl.program_id(1)

    @pl.when(e == 0)
    def _():
        xb = x_ref[...].astype(jnp.bfloat16)
        logits = jnp.dot(xb, rw_ref[...].astype(jnp.bfloat16),
                         preferred_element_type=jnp.float32)
        probs_ref[...] = jax.nn.softmax(logits + rb_ref[...], axis=-1)
        sh = jax.nn.gelu(
            jnp.dot(xb, sw1_ref[...], preferred_element_type=jnp.float32)
            + sb1_ref[...])
        out_ref[...] = jnp.dot(sh.astype(jnp.bfloat16), sw2_ref[...],
                               preferred_element_type=jnp.float32) + sb2_ref[...]

    t = emb_ref[...].astype(jnp.bfloat16)
    h = jnp.dot(t, w1_ref[0].astype(jnp.bfloat16),
                preferred_element_type=jnp.float32)
    h = jax.nn.gelu(h + b1_ref[0])

    sel = jax.lax.broadcasted_iota(jnp.int32, (1, 8), 1) == e
    gate = jnp.sum(probs_ref[...] * sel.astype(jnp.float32), axis=-1,
                   keepdims=True)
    hg = (h * gate).astype(jnp.bfloat16)
    contrib = jnp.dot(hg, w2_ref[0].astype(jnp.bfloat16),
                      preferred_element_type=jnp.float32)
    out_ref[...] += contrib + gate * b2_ref[0]


def kernel(x, embedding_tokens, router_W, router_b, expert_W1, expert_b1,
           expert_W2, expert_b2, shared_W1, shared_b1, shared_W2, shared_b2):
    B, S, D = x.shape
    F = shared_W1.shape[1]
    E = router_W.shape[1]
    n = B * S

    out = pl.pallas_call(
        _moe_body,
        grid=(n // _TB, E),
        in_specs=[
            pl.BlockSpec((_TB, D), lambda i, e: (i, 0)),
            pl.BlockSpec((_TB, D), lambda i, e: (i, 0)),
            pl.BlockSpec((D, E), lambda i, e: (0, 0)),
            pl.BlockSpec((1, E), lambda i, e: (0, 0)),
            pl.BlockSpec((1, D, F), lambda i, e: (e, 0, 0)),
            pl.BlockSpec((1, 1, F), lambda i, e: (e, 0, 0)),
            pl.BlockSpec((1, F, D), lambda i, e: (e, 0, 0)),
            pl.BlockSpec((1, 1, D), lambda i, e: (e, 0, 0)),
            pl.BlockSpec((D, F), lambda i, e: (0, 0)),
            pl.BlockSpec((1, F), lambda i, e: (0, 0)),
            pl.BlockSpec((F, D), lambda i, e: (0, 0)),
            pl.BlockSpec((1, D), lambda i, e: (0, 0)),
        ],
        out_specs=pl.BlockSpec((_TB, D), lambda i, e: (i, 0)),
        out_shape=jax.ShapeDtypeStruct((n, D), jnp.float32),
        scratch_shapes=[pltpu.VMEM((_TB, E), jnp.float32)],
        compiler_params=pltpu.CompilerParams(
            dimension_semantics=("parallel", "arbitrary")),
    )(x.reshape(n, D), embedding_tokens.reshape(n, D),
      router_W, router_b[None, :],
      expert_W1, expert_b1[:, None, :], expert_W2, expert_b2[:, None, :],
      shared_W1.astype(jnp.bfloat16), shared_b1[None, :],
      shared_W2.astype(jnp.bfloat16), shared_b2[None, :])

    return out.reshape(B, S, D)


# speedup vs baseline: 1.1937x; 1.0359x over previous
import jax
import jax.numpy as jnp
from jax.experimental import pallas as pl
from jax.experimental.pallas import tpu as pltpu

_TB = 512


def _moe_body(x_ref, emb_ref, rw_ref, rb_ref, w1_ref, b1_ref, w2_ref, b2_ref,
              sw1_ref, sb1_ref, sw2_ref, sb2_ref, out_ref, probs_ref):
    e = pl.program_id(1)

    @pl.when(e == 0)
    def _():
        xb = x_ref[...].astype(jnp.bfloat16)
        logits = jnp.dot(xb, rw_ref[...].astype(jnp.bfloat16),
                         preferred_element_type=jnp.float32)
        probs_ref[...] = jax.nn.softmax(logits + rb_ref[...], axis=-1)
        sh = jnp.dot(xb, sw1_ref[...],
                     preferred_element_type=jnp.float32).astype(jnp.bfloat16)
        sh = jax.nn.gelu(sh + sb1_ref[...].astype(jnp.bfloat16))
        out_ref[...] = jnp.dot(sh, sw2_ref[...],
                               preferred_element_type=jnp.float32) + sb2_ref[...]

    t = emb_ref[...].astype(jnp.bfloat16)
    h = jnp.dot(t, w1_ref[0].astype(jnp.bfloat16),
                preferred_element_type=jnp.float32).astype(jnp.bfloat16)
    h = jax.nn.gelu(h + b1_ref[0].astype(jnp.bfloat16))

    sel = jax.lax.broadcasted_iota(jnp.int32, (1, 8), 1) == e
    gate = jnp.sum(probs_ref[...] * sel.astype(jnp.float32), axis=-1,
                   keepdims=True)
    hg = h * gate.astype(jnp.bfloat16)
    contrib = jnp.dot(hg, w2_ref[0].astype(jnp.bfloat16),
                      preferred_element_type=jnp.float32)
    out_ref[...] += contrib + gate * b2_ref[0]


def kernel(x, embedding_tokens, router_W, router_b, expert_W1, expert_b1,
           expert_W2, expert_b2, shared_W1, shared_b1, shared_W2, shared_b2):
    B, S, D = x.shape
    F = shared_W1.shape[1]
    E = router_W.shape[1]
    n = B * S

    out = pl.pallas_call(
        _moe_body,
        grid=(n // _TB, E),
        in_specs=[
            pl.BlockSpec((_TB, D), lambda i, e: (i, 0)),
            pl.BlockSpec((_TB, D), lambda i, e: (i, 0)),
            pl.BlockSpec((D, E), lambda i, e: (0, 0)),
            pl.BlockSpec((1, E), lambda i, e: (0, 0)),
            pl.BlockSpec((1, D, F), lambda i, e: (e, 0, 0)),
            pl.BlockSpec((1, 1, F), lambda i, e: (e, 0, 0)),
            pl.BlockSpec((1, F, D), lambda i, e: (e, 0, 0)),
            pl.BlockSpec((1, 1, D), lambda i, e: (e, 0, 0)),
            pl.BlockSpec((D, F), lambda i, e: (0, 0)),
            pl.BlockSpec((1, F), lambda i, e: (0, 0)),
            pl.BlockSpec((F, D), lambda i, e: (0, 0)),
            pl.BlockSpec((1, D), lambda i, e: (0, 0)),
        ],
        out_specs=pl.BlockSpec((_TB, D), lambda i, e: (i, 0)),
        out_shape=jax.ShapeDtypeStruct((n, D), jnp.float32),
        scratch_shapes=[pltpu.VMEM((_TB, E), jnp.float32)],
        compiler_params=pltpu.CompilerParams(
            dimension_semantics=("parallel", "arbitrary")),
    )(x.reshape(n, D), embedding_tokens.reshape(n, D),
      router_W, router_b[None, :],
      expert_W1, expert_b1[:, None, :], expert_W2, expert_b2[:, None, :],
      shared_W1.astype(jnp.bfloat16), shared_b1[None, :],
      shared_W2.astype(jnp.bfloat16), shared_b2[None, :])

    return out.reshape(B, S, D)


# speedup vs baseline: 1.4280x; 1.1963x over previous
import jax
import jax.numpy as jnp
from jax.experimental import pallas as pl
from jax.experimental.pallas import tpu as pltpu

_TB = 1024


def _moe_body(x_ref, emb_ref, rw_ref, rb_ref, w1_ref, b1_ref, w2_ref, b2_ref,
              sw1_ref, sb1_ref, sw2_ref, sb2_ref, out_ref, probs_ref):
    e = pl.program_id(1)

    @pl.when(e == 0)
    def _():
        xb = x_ref[...].astype(jnp.bfloat16)
        logits = jnp.dot(xb, rw_ref[...].astype(jnp.bfloat16),
                         preferred_element_type=jnp.float32)
        probs_ref[...] = jax.nn.softmax(logits + rb_ref[...], axis=-1)
        sh = jnp.dot(xb, sw1_ref[...],
                     preferred_element_type=jnp.float32).astype(jnp.bfloat16)
        sh = jax.nn.gelu(sh + sb1_ref[...].astype(jnp.bfloat16))
        out_ref[...] = jnp.dot(sh, sw2_ref[...],
                               preferred_element_type=jnp.float32) + sb2_ref[...]

    t = emb_ref[...].astype(jnp.bfloat16)
    h = jnp.dot(t, w1_ref[0].astype(jnp.bfloat16),
                preferred_element_type=jnp.float32).astype(jnp.bfloat16)
    h = jax.nn.gelu(h + b1_ref[0].astype(jnp.bfloat16))

    sel = jax.lax.broadcasted_iota(jnp.int32, (1, 8), 1) == e
    gate = jnp.sum(probs_ref[...] * sel.astype(jnp.float32), axis=-1,
                   keepdims=True)
    hg = h * gate.astype(jnp.bfloat16)
    contrib = jnp.dot(hg, w2_ref[0].astype(jnp.bfloat16),
                      preferred_element_type=jnp.float32)
    out_ref[...] += contrib + gate * b2_ref[0]


def kernel(x, embedding_tokens, router_W, router_b, expert_W1, expert_b1,
           expert_W2, expert_b2, shared_W1, shared_b1, shared_W2, shared_b2):
    B, S, D = x.shape
    F = shared_W1.shape[1]
    E = router_W.shape[1]
    n = B * S

    out = pl.pallas_call(
        _moe_body,
        grid=(n // _TB, E),
        in_specs=[
            pl.BlockSpec((_TB, D), lambda i, e: (i, 0)),
            pl.BlockSpec((_TB, D), lambda i, e: (i, 0)),
            pl.BlockSpec((D, E), lambda i, e: (0, 0)),
            pl.BlockSpec((1, E), lambda i, e: (0, 0)),
            pl.BlockSpec((1, D, F), lambda i, e: (e, 0, 0)),
            pl.BlockSpec((1, 1, F), lambda i, e: (e, 0, 0)),
            pl.BlockSpec((1, F, D), lambda i, e: (e, 0, 0)),
            pl.BlockSpec((1, 1, D), lambda i, e: (e, 0, 0)),
            pl.BlockSpec((D, F), lambda i, e: (0, 0)),
            pl.BlockSpec((1, F), lambda i, e: (0, 0)),
            pl.BlockSpec((F, D), lambda i, e: (0, 0)),
            pl.BlockSpec((1, D), lambda i, e: (0, 0)),
        ],
        out_specs=pl.BlockSpec((_TB, D), lambda i, e: (i, 0)),
        out_shape=jax.ShapeDtypeStruct((n, D), jnp.float32),
        scratch_shapes=[pltpu.VMEM((_TB, E), jnp.float32)],
        compiler_params=pltpu.CompilerParams(
            dimension_semantics=("parallel", "arbitrary")),
    )(x.reshape(n, D), embedding_tokens.reshape(n, D),
      router_W, router_b[None, :],
      expert_W1, expert_b1[:, None, :], expert_W2, expert_b2[:, None, :],
      shared_W1.astype(jnp.bfloat16), shared_b1[None, :],
      shared_W2.astype(jnp.bfloat16), shared_b2[None, :])

    return out.reshape(B, S, D)
